# baseline (speedup 1.0000x reference)
import functools

import jax
import jax.numpy as jnp
from jax import lax
from jax.experimental import pallas as pl
from jax.experimental.pallas import tpu as pltpu
from jax.experimental.pallas import tpu_sc as plsc

_NCB = 4
_K = 8192
_C = 64
_B = 64
_HW = 1024
_KT = 1024
_N_PER_CB = _B * _HW


def _argmin_body(e_ref, x_ref, amin_ref, flat_ref, eaug_ref, m_ref):
    i = pl.program_id(0)
    b = pl.program_id(1)

    @pl.when(b == 0)
    def _build_e2():
        e = e_ref[0]
        eaug_ref[...] = jnp.sum(e * e, axis=1, keepdims=True)

    x = x_ref[0, 0]
    xneg = -2.0 * x
    av = jnp.full((8, _HW), jnp.inf, dtype=jnp.float32)
    ai = jnp.zeros((8, _HW), dtype=jnp.int32)
    for kt in range(_K // _KT):
        ek = e_ref[0, pl.ds(kt * _KT, _KT), :]
        m_ref[kt % 2] = jax.lax.dot_general(
            ek, xneg, (((1,), (0,)), ((), ())),
            preferred_element_type=jnp.float32)

        def step(v, carry, kt=kt):
            cv, ci = carry
            d = (m_ref[kt % 2, pl.ds(v * 8, 8), :]
                 + eaug_ref[pl.ds(kt * _KT + v * 8, 8), :])
            upd = d < cv
            sidx = v + kt * (_KT // 8)
            return (jnp.where(upd, d, cv), jnp.where(upd, sidx, ci))

        av, ai = lax.fori_loop(0, _KT // 8, step, (av, ai), unroll=8)
    ai = ai * 8 + jax.lax.broadcasted_iota(jnp.int32, (8, _HW), 0)

    def merge(pair_a, pair_b):
        va, ia = pair_a
        vb, ib = pair_b
        t = (vb < va) | ((vb == va) & (ib < ia))
        return jnp.where(t, vb, va), jnp.where(t, ib, ia)

    v4, i4 = merge((av[0:4], ai[0:4]), (av[4:8], ai[4:8]))
    v2, i2 = merge((v4[0:2], i4[0:2]), (v4[2:4], i4[2:4]))
    _, i1 = merge((v2[0:1], i2[0:1]), (v2[1:2], i2[1:2]))
    besti = i1[0]
    amin_ref[0, 0, 0, :] = besti
    flat_ref[0, 0, 0, :] = besti + i * _K


def _run_argmin(z4, codebooks):
    return pl.pallas_call(
        _argmin_body,
        grid=(_NCB, _B),
        in_specs=[
            pl.BlockSpec((1, _K, _C), lambda i, b: (i, 0, 0)),
            pl.BlockSpec((1, 1, _C, _HW), lambda i, b: (b, i, 0, 0)),
        ],
        out_specs=[
            pl.BlockSpec((1, 1, 1, _HW), lambda i, b: (i, b, 0, 0)),
            pl.BlockSpec((1, 1, 1, _HW), lambda i, b: (i, b, 0, 0)),
        ],
        out_shape=[
            jax.ShapeDtypeStruct((_NCB, _B, 1, _HW), jnp.int32),
            jax.ShapeDtypeStruct((_NCB, _B, 1, _HW), jnp.int32),
        ],
        scratch_shapes=[pltpu.VMEM((_K, 1), jnp.float32),
                        pltpu.VMEM((2, _KT, _HW), jnp.float32)],
    )(codebooks, z4)


_NW = 32
_NIDX = _NCB * _B * _HW
_PER_W = _NIDX // _NW
_CHUNK = 128
_NCHUNK = _PER_W // _CHUNK
_KFLAT = _NCB * _K


def _sc_gather_bincount(cb_flat, flat_idx):
    mesh = plsc.VectorSubcoreMesh(core_axis_name="c", subcore_axis_name="s")

    @functools.partial(
        pl.kernel, mesh=mesh,
        out_type=[
            jax.ShapeDtypeStruct((_NIDX, 128), jnp.float32),
            jax.ShapeDtypeStruct((2, _KFLAT), jnp.float32),
        ],
        scratch_types=[
            pltpu.VMEM((_CHUNK,), jnp.int32),
            pltpu.VMEM((_CHUNK, 128), jnp.float32),
            pltpu.VMEM((_CHUNK,), jnp.float32),
            pltpu.VMEM((2048,), jnp.float32),
            pltpu.VMEM_SHARED((_KFLAT,), jnp.float32),
            pltpu.SemaphoreType.DMA,
        ],
    )
    def sc_kernel(cb_hbm, idx_hbm, out_hbm, cnt_hbm,
                  idx_v, rows_v, ones_v, zeros_v, hist_sh, sem):
        cid = lax.axis_index("c")
        sid = lax.axis_index("s")
        base = (sid * 2 + cid) * _PER_W
        for j in range(_CHUNK // 16):
            ones_v[pl.ds(j * 16, 16)] = jnp.ones((16,), jnp.float32)

        @pl.when(sid == 0)
        def _zero_hist():
            for j in range(128):
                zeros_v[pl.ds(j * 16, 16)] = jnp.zeros((16,), jnp.float32)
            for j in range(_KFLAT // 2048):
                pltpu.sync_copy(zeros_v, hist_sh.at[pl.ds(j * 2048, 2048)])

        plsc.subcore_barrier()

        def chunk(c, carry):
            off = base + c * _CHUNK
            pltpu.sync_copy(idx_hbm.at[pl.ds(off, _CHUNK)], idx_v)
            pltpu.async_copy(cb_hbm.at[idx_v], rows_v, sem).wait()
            pltpu.sync_copy(rows_v, out_hbm.at[pl.ds(off, _CHUNK)])
            pltpu.sync_copy(ones_v, hist_sh.at[idx_v], add=True)
            return carry

        lax.fori_loop(0, _NCHUNK, chunk, 0)
        plsc.subcore_barrier()

        @pl.when(sid == 0)
        def _dump_hist():
            pltpu.sync_copy(hist_sh, cnt_hbm.at[cid])

    return sc_kernel(cb_flat, flat_idx)


def _assemble_body(g_ref, z_ref, zq_ref, r2_ref):
    g = g_ref[0, 0, :, : _C]
    q = jnp.swapaxes(g, 0, 1)
    x = z_ref[0, 0]
    zq_ref[0, 0] = q
    r = q - x
    r2_ref[0, 0, 0, :] = jnp.sum(r * r, axis=0)


def _run_assemble(gathered, z4):
    return pl.pallas_call(
        _assemble_body,
        grid=(_NCB, _B),
        in_specs=[
            pl.BlockSpec((1, 1, _HW, 128), lambda i, b: (i, b, 0, 0)),
            pl.BlockSpec((1, 1, _C, _HW), lambda i, b: (b, i, 0, 0)),
        ],
        out_specs=[
            pl.BlockSpec((1, 1, _C, _HW), lambda i, b: (b, i, 0, 0)),
            pl.BlockSpec((1, 1, 1, _HW), lambda i, b: (i, b, 0, 0)),
        ],
        out_shape=[
            jax.ShapeDtypeStruct((_B, _NCB, _C, _HW), jnp.float32),
            jax.ShapeDtypeStruct((_NCB, _B, 1, _HW), jnp.float32),
        ],
    )(gathered, z4)


def _final_body(r2_ref, cnt_ref, diff_ref, ppl_ref):
    r2 = r2_ref[...]
    diff_ref[...] = (jnp.sum(r2) / jnp.float32(_NCB * _N_PER_CB * _C)).reshape(1, 1)
    counts = cnt_ref[0] + cnt_ref[1]
    total = jnp.maximum(jnp.sum(counts, axis=1, keepdims=True), 1.0)
    probs = counts / total
    ent = jnp.where(probs > 0, probs * jnp.log(probs + 1e-10), 0.0)
    ppl_ref[0, :] = jnp.exp(-jnp.sum(ent, axis=1))


def _run_final(r2, counts):
    return pl.pallas_call(
        _final_body,
        out_shape=[
            jax.ShapeDtypeStruct((1, 1), jnp.float32),
            jax.ShapeDtypeStruct((1, _NCB), jnp.float32),
        ],
    )(r2, counts)


def kernel(z_e, codebooks):
    z4 = z_e.reshape(_B, _NCB, _C, _HW)
    amin, flat = _run_argmin(z4, codebooks)
    flat_idx = flat.reshape(-1)
    cb_flat = codebooks.reshape(_NCB * _K, _C)
    cb_pad = jnp.concatenate([cb_flat, jnp.zeros_like(cb_flat)], axis=1)
    rows, counts2 = _sc_gather_bincount(cb_pad, flat_idx)
    gathered = rows.reshape(_NCB, _B, _HW, 128)
    zq4, r2 = _run_assemble(gathered, z4)
    diff, ppls = _run_final(r2.reshape(_NCB, _B, _HW),
                            counts2.reshape(2, _NCB, _K))
    z_q = zq4.reshape(_B, _NCB * _C, 32, 32)
    argmins = amin.reshape(_NCB, _B, 32, 32)
    return (z_q, diff[0, 0], argmins, ppls[0])


# speedup vs baseline: 1.8313x; 1.8313x over previous
import functools

import jax
import jax.numpy as jnp
from jax import lax
from jax.experimental import pallas as pl
from jax.experimental.pallas import tpu as pltpu
from jax.experimental.pallas import tpu_sc as plsc

_NCB = 4
_K = 8192
_C = 64
_B = 64
_HW = 1024
_KT = 1024
_N_PER_CB = _B * _HW


def _argmin_body(e_ref, x_ref, amin_ref, flat_ref, eaug_ref):
    i = pl.program_id(0)
    b = pl.program_id(1)

    @pl.when(b == 0)
    def _build_e2():
        e = e_ref[0]
        eaug_ref[...] = jnp.sum(e * e, axis=1, keepdims=True)

    x = x_ref[0, 0]
    xneg = -2.0 * x
    best = jnp.full((_HW,), jnp.inf, dtype=jnp.float32)
    besti = jnp.zeros((_HW,), dtype=jnp.int32)
    for kt in range(_K // _KT):
        ek = e_ref[0, pl.ds(kt * _KT, _KT), :]
        e2 = eaug_ref[pl.ds(kt * _KT, _KT), :]
        m = jax.lax.dot_general(
            ek, xneg, (((1,), (0,)), ((), ())),
            preferred_element_type=jnp.float32)
        d = m + e2
        mv = jnp.min(d, axis=0)
        ii = jnp.where(d <= mv[None, :],
                       jax.lax.broadcasted_iota(jnp.int32, d.shape, 0),
                       jnp.int32(2 ** 30))
        mi = jnp.min(ii, axis=0) + kt * _KT
        upd = mv < best
        besti = jnp.where(upd, mi, besti)
        best = jnp.where(upd, mv, best)
    amin_ref[0, 0, 0, :] = besti
    flat_ref[0, 0, 0, :] = besti + i * _K


def _run_argmin(z4, codebooks):
    return pl.pallas_call(
        _argmin_body,
        grid=(_NCB, _B),
        in_specs=[
            pl.BlockSpec((1, _K, _C), lambda i, b: (i, 0, 0)),
            pl.BlockSpec((1, 1, _C, _HW), lambda i, b: (b, i, 0, 0)),
        ],
        out_specs=[
            pl.BlockSpec((1, 1, 1, _HW), lambda i, b: (i, b, 0, 0)),
            pl.BlockSpec((1, 1, 1, _HW), lambda i, b: (i, b, 0, 0)),
        ],
        out_shape=[
            jax.ShapeDtypeStruct((_NCB, _B, 1, _HW), jnp.int32),
            jax.ShapeDtypeStruct((_NCB, _B, 1, _HW), jnp.int32),
        ],
        scratch_shapes=[pltpu.VMEM((_K, 1), jnp.float32)],
    )(codebooks, z4)


_NW = 32
_NIDX = _NCB * _B * _HW
_PER_W = _NIDX // _NW
_CHUNK = 128
_NCHUNK = _PER_W // _CHUNK
_KFLAT = _NCB * _K


def _sc_gather_bincount(cb_flat, flat_idx):
    mesh = plsc.VectorSubcoreMesh(core_axis_name="c", subcore_axis_name="s")

    @functools.partial(
        pl.kernel, mesh=mesh,
        out_type=[
            jax.ShapeDtypeStruct((_NIDX, 128), jnp.float32),
            jax.ShapeDtypeStruct((2, _KFLAT), jnp.float32),
        ],
        scratch_types=[
            pltpu.VMEM((_CHUNK,), jnp.int32),
            pltpu.VMEM((_CHUNK, 128), jnp.float32),
            pltpu.VMEM((_CHUNK,), jnp.float32),
            pltpu.VMEM((2048,), jnp.float32),
            pltpu.VMEM_SHARED((_KFLAT,), jnp.float32),
            pltpu.SemaphoreType.DMA,
        ],
    )
    def sc_kernel(cb_hbm, idx_hbm, out_hbm, cnt_hbm,
                  idx_v, rows_v, ones_v, zeros_v, hist_sh, sem):
        cid = lax.axis_index("c")
        sid = lax.axis_index("s")
        base = (sid * 2 + cid) * _PER_W
        for j in range(_CHUNK // 16):
            ones_v[pl.ds(j * 16, 16)] = jnp.ones((16,), jnp.float32)

        @pl.when(sid == 0)
        def _zero_hist():
            for j in range(128):
                zeros_v[pl.ds(j * 16, 16)] = jnp.zeros((16,), jnp.float32)
            for j in range(_KFLAT // 2048):
                pltpu.sync_copy(zeros_v, hist_sh.at[pl.ds(j * 2048, 2048)])

        plsc.subcore_barrier()

        def chunk(c, carry):
            off = base + c * _CHUNK
            pltpu.sync_copy(idx_hbm.at[pl.ds(off, _CHUNK)], idx_v)
            pltpu.async_copy(cb_hbm.at[idx_v], rows_v, sem).wait()
            pltpu.sync_copy(rows_v, out_hbm.at[pl.ds(off, _CHUNK)])
            pltpu.sync_copy(ones_v, hist_sh.at[idx_v], add=True)
            return carry

        lax.fori_loop(0, _NCHUNK, chunk, 0)
        plsc.subcore_barrier()

        @pl.when(sid == 0)
        def _dump_hist():
            pltpu.sync_copy(hist_sh, cnt_hbm.at[cid])

    return sc_kernel(cb_flat, flat_idx)


def _assemble_body(g_ref, z_ref, zq_ref, r2_ref):
    g = g_ref[0, 0, :, : _C]
    q = jnp.swapaxes(g, 0, 1)
    x = z_ref[0, 0]
    zq_ref[0, 0] = q
    r = q - x
    r2_ref[0, 0, 0, :] = jnp.sum(r * r, axis=0)


def _run_assemble(gathered, z4):
    return pl.pallas_call(
        _assemble_body,
        grid=(_NCB, _B),
        in_specs=[
            pl.BlockSpec((1, 1, _HW, 128), lambda i, b: (i, b, 0, 0)),
            pl.BlockSpec((1, 1, _C, _HW), lambda i, b: (b, i, 0, 0)),
        ],
        out_specs=[
            pl.BlockSpec((1, 1, _C, _HW), lambda i, b: (b, i, 0, 0)),
            pl.BlockSpec((1, 1, 1, _HW), lambda i, b: (i, b, 0, 0)),
        ],
        out_shape=[
            jax.ShapeDtypeStruct((_B, _NCB, _C, _HW), jnp.float32),
            jax.ShapeDtypeStruct((_NCB, _B, 1, _HW), jnp.float32),
        ],
    )(gathered, z4)


def _final_body(r2_ref, cnt_ref, diff_ref, ppl_ref):
    r2 = r2_ref[...]
    diff_ref[...] = (jnp.sum(r2) / jnp.float32(_NCB * _N_PER_CB * _C)).reshape(1, 1)
    counts = cnt_ref[0] + cnt_ref[1]
    total = jnp.maximum(jnp.sum(counts, axis=1, keepdims=True), 1.0)
    probs = counts / total
    ent = jnp.where(probs > 0, probs * jnp.log(probs + 1e-10), 0.0)
    ppl_ref[0, :] = jnp.exp(-jnp.sum(ent, axis=1))


def _run_final(r2, counts):
    return pl.pallas_call(
        _final_body,
        out_shape=[
            jax.ShapeDtypeStruct((1, 1), jnp.float32),
            jax.ShapeDtypeStruct((1, _NCB), jnp.float32),
        ],
    )(r2, counts)


def kernel(z_e, codebooks):
    z4 = z_e.reshape(_B, _NCB, _C, _HW)
    amin, flat = _run_argmin(z4, codebooks)
    flat_idx = flat.reshape(-1)
    cb_flat = codebooks.reshape(_NCB * _K, _C)
    cb_pad = jnp.concatenate([cb_flat, jnp.zeros_like(cb_flat)], axis=1)
    rows, counts2 = _sc_gather_bincount(cb_pad, flat_idx)
    gathered = rows.reshape(_NCB, _B, _HW, 128)
    zq4, r2 = _run_assemble(gathered, z4)
    diff, ppls = _run_final(r2.reshape(_NCB, _B, _HW),
                            counts2.reshape(2, _NCB, _K))
    z_q = zq4.reshape(_B, _NCB * _C, 32, 32)
    argmins = amin.reshape(_NCB, _B, 32, 32)
    return (z_q, diff[0, 0], argmins, ppls[0])
